# baseline (device time: 60175 ns/iter reference)
import jax
import jax.numpy as jnp
from jax import lax
from jax.experimental import pallas as pl
from jax.experimental.pallas import tpu as pltpu

T = 256
D = 512
VP = 4096
V = 2 * VP


def kernel(x, W):
    def body(x_ref, w_ref, out_ref, send_sem, recv_sem):
        my_x = lax.axis_index("x")
        my_y = lax.axis_index("y")
        nbr = (1 - my_x, my_y)

        barrier_sem = pltpu.get_barrier_semaphore()
        pl.semaphore_signal(
            barrier_sem, inc=1, device_id=nbr,
            device_id_type=pl.DeviceIdType.MESH,
        )
        pl.semaphore_wait(barrier_sem, 1)

        logits = jnp.dot(
            x_ref[...], w_ref[...], preferred_element_type=jnp.float32
        )

        def exchange(lo):
            out_ref[:, lo:lo + VP] = logits
            rdma = pltpu.make_async_remote_copy(
                src_ref=out_ref.at[:, pl.ds(lo, VP)],
                dst_ref=out_ref.at[:, pl.ds(lo, VP)],
                send_sem=send_sem,
                recv_sem=recv_sem,
                device_id=nbr,
                device_id_type=pl.DeviceIdType.MESH,
            )
            rdma.start()
            rdma.wait()

        @pl.when(my_x == 0)
        def _():
            exchange(0)

        @pl.when(my_x == 1)
        def _():
            exchange(VP)

        full = out_ref[...]
        m = jnp.max(full, axis=1, keepdims=True)
        e = jnp.exp(full - m)
        out_ref[...] = e / jnp.sum(e, axis=1, keepdims=True)

    return pl.pallas_call(
        body,
        out_shape=jax.ShapeDtypeStruct((T, V), jnp.float32),
        in_specs=[
            pl.BlockSpec(memory_space=pltpu.VMEM),
            pl.BlockSpec(memory_space=pltpu.VMEM),
        ],
        out_specs=pl.BlockSpec(memory_space=pltpu.VMEM),
        scratch_shapes=[
            pltpu.SemaphoreType.DMA,
            pltpu.SemaphoreType.DMA,
        ],
        compiler_params=pltpu.CompilerParams(collective_id=0),
    )(x, W)


# device time: 57603 ns/iter; 1.0447x vs baseline; 1.0447x over previous
import jax
import jax.numpy as jnp
from jax import lax
from jax.experimental import pallas as pl
from jax.experimental.pallas import tpu as pltpu

T = 256
D = 512
VP = 4096
V = 2 * VP
CHUNK = 512
NCH = VP // CHUNK


def kernel(x, W):
    def body(x_ref, w_ref, out_ref, send_sems, recv_sems):
        my_x = lax.axis_index("x")
        my_y = lax.axis_index("y")
        nbr = (1 - my_x, my_y)

        barrier_sem = pltpu.get_barrier_semaphore()
        pl.semaphore_signal(
            barrier_sem, inc=1, device_id=nbr,
            device_id_type=pl.DeviceIdType.MESH,
        )
        pl.semaphore_wait(barrier_sem, 1)

        xv = x_ref[...]

        def run(my_lo):
            nbr_lo = VP - my_lo
            rdmas = []
            s_loc = jnp.zeros((T, 1), jnp.float32)
            for c in range(NCH):
                lo = my_lo + c * CHUNK
                logits_c = jnp.dot(
                    xv, w_ref[:, c * CHUNK:(c + 1) * CHUNK],
                    preferred_element_type=jnp.float32,
                )
                e_c = jnp.exp(logits_c)
                s_loc = s_loc + jnp.sum(e_c, axis=1, keepdims=True)
                out_ref[:, lo:lo + CHUNK] = e_c
                rdma = pltpu.make_async_remote_copy(
                    src_ref=out_ref.at[:, pl.ds(lo, CHUNK)],
                    dst_ref=out_ref.at[:, pl.ds(lo, CHUNK)],
                    send_sem=send_sems.at[c],
                    recv_sem=recv_sems.at[c],
                    device_id=nbr,
                    device_id_type=pl.DeviceIdType.MESH,
                )
                rdma.start()
                rdmas.append(rdma)

            s_nbr = jnp.zeros((T, 1), jnp.float32)
            for c in range(NCH):
                rdmas[c].wait_recv()
                lo = nbr_lo + c * CHUNK
                s_nbr = s_nbr + jnp.sum(
                    out_ref[:, lo:lo + CHUNK], axis=1, keepdims=True
                )
            for c in range(NCH):
                rdmas[c].wait_send()

            inv = 1.0 / (s_loc + s_nbr)
            out_ref[...] = out_ref[...] * inv

        @pl.when(my_x == 0)
        def _():
            run(0)

        @pl.when(my_x == 1)
        def _():
            run(VP)

    return pl.pallas_call(
        body,
        out_shape=jax.ShapeDtypeStruct((T, V), jnp.float32),
        in_specs=[
            pl.BlockSpec(memory_space=pltpu.VMEM),
            pl.BlockSpec(memory_space=pltpu.VMEM),
        ],
        out_specs=pl.BlockSpec(memory_space=pltpu.VMEM),
        scratch_shapes=[
            pltpu.SemaphoreType.DMA((NCH,)),
            pltpu.SemaphoreType.DMA((NCH,)),
        ],
        compiler_params=pltpu.CompilerParams(collective_id=0),
    )(x, W)
